# baseline (speedup 1.0000x reference)
import jax
import jax.numpy as jnp
from jax.experimental import pallas as pl
from jax.experimental.pallas import tpu as pltpu

TILE = 1024


def _router_tile(t_ref, hid_ref, feat_ref, w1h_ref, wcomb_ref, bcomb_ref,
                 w2_ref, b2_ref, w_out_ref, l_out_ref):
    acc = jnp.dot(hid_ref[...], w1h_ref[...],
                  preferred_element_type=jnp.float32)
    acc = acc + jnp.dot(feat_ref[...], wcomb_ref[...],
                        preferred_element_type=jnp.float32)
    h = jnp.maximum(acc + bcomb_ref[...], 0.0)
    logits = jnp.dot(h, w2_ref[...],
                     preferred_element_type=jnp.float32) + b2_ref[...]
    l_out_ref[...] = logits
    logits_t = logits.T

    scaled = logits_t * t_ref[0, 0]
    n_stages = scaled.shape[0]
    idx = jax.lax.broadcasted_iota(jnp.int32, scaled.shape, 0)
    m1 = jnp.max(scaled, axis=0, keepdims=True)
    first = jnp.min(jnp.where(scaled == m1, idx, n_stages), axis=0,
                    keepdims=True)
    m2 = jnp.max(jnp.where(idx == first, -jnp.inf, scaled), axis=0,
                 keepdims=True)
    keep = scaled >= m2
    e = jnp.where(keep, jnp.exp(scaled - m1), 0.0)
    w_out_ref[...] = (e / jnp.sum(e, axis=0, keepdims=True)).T


def kernel(hidden, feat, W_feat, b_feat, W1, b1, W2, b2, temperature):
    n_tokens, d_model = hidden.shape
    n_feat = feat.shape[1]
    d_hid = W1.shape[1]
    n_stages = W2.shape[1]
    w_comb = jnp.dot(W_feat, W1[d_model:, :],
                     precision=jax.lax.Precision.HIGHEST)
    b_comb = (jnp.dot(b_feat, W1[d_model:, :],
                      precision=jax.lax.Precision.HIGHEST) + b1).reshape(1, -1)
    inv_t = (1.0 / jnp.asarray(temperature, jnp.float32)).reshape(1, 1)

    grid = (n_tokens // TILE,)
    out = pl.pallas_call(
        _router_tile,
        grid=grid,
        in_specs=[
            pl.BlockSpec(memory_space=pltpu.SMEM),
            pl.BlockSpec((TILE, d_model), lambda i: (i, 0)),
            pl.BlockSpec((TILE, n_feat), lambda i: (i, 0)),
            pl.BlockSpec((d_model, d_hid), lambda i: (0, 0)),
            pl.BlockSpec((n_feat, d_hid), lambda i: (0, 0)),
            pl.BlockSpec((1, d_hid), lambda i: (0, 0)),
            pl.BlockSpec((d_hid, n_stages), lambda i: (0, 0)),
            pl.BlockSpec((1, n_stages), lambda i: (0, 0)),
        ],
        out_specs=[
            pl.BlockSpec((TILE, n_stages), lambda i: (i, 0)),
            pl.BlockSpec((TILE, n_stages), lambda i: (i, 0)),
        ],
        out_shape=[
            jax.ShapeDtypeStruct((n_tokens, n_stages), jnp.float32),
            jax.ShapeDtypeStruct((n_tokens, n_stages), jnp.float32),
        ],
        compiler_params=pltpu.CompilerParams(
            dimension_semantics=("parallel",)),
    )(inv_t, hidden, feat, W1[:d_model, :], w_comb, b_comb, W2,
      b2.reshape(1, -1))
    return out[0], out[1]


# speedup vs baseline: 1.0456x; 1.0456x over previous
import jax
import jax.numpy as jnp
from jax.experimental import pallas as pl
from jax.experimental.pallas import tpu as pltpu
from functools import partial

TILE = 1024


def _router_tile(t_ref, hid_ref, feat_ref, wf_ref, bf_ref, w1_ref, b1_ref,
                 w2_ref, b2_ref, w_out_ref, l_out_ref, *, d_model):
    femb = jnp.dot(feat_ref[...], wf_ref[...],
                   preferred_element_type=jnp.float32) + bf_ref[...]
    acc = jnp.dot(femb, w1_ref[d_model:, :],
                  preferred_element_type=jnp.float32)
    acc = acc + jnp.dot(hid_ref[...], w1_ref[0:d_model, :],
                        preferred_element_type=jnp.float32)
    h = jnp.maximum(acc + b1_ref[...], 0.0)
    logits = jnp.dot(h, w2_ref[...],
                     preferred_element_type=jnp.float32) + b2_ref[...]
    l_out_ref[...] = logits

    scaled = logits.T / t_ref[0, 0]
    n_stages = scaled.shape[0]
    idx = jax.lax.broadcasted_iota(jnp.int32, scaled.shape, 0)
    m1 = jnp.max(scaled, axis=0, keepdims=True)
    first = jnp.min(jnp.where(scaled == m1, idx, n_stages), axis=0,
                    keepdims=True)
    m2 = jnp.max(jnp.where(idx == first, -jnp.inf, scaled), axis=0,
                 keepdims=True)
    keep = scaled >= m2
    e = jnp.where(keep, jnp.exp(scaled - m1), 0.0)
    w_out_ref[...] = (e / jnp.sum(e, axis=0, keepdims=True)).T


def kernel(hidden, feat, W_feat, b_feat, W1, b1, W2, b2, temperature):
    n_tokens, d_model = hidden.shape
    n_feat, d_femb = W_feat.shape
    d_hid = W1.shape[1]
    n_stages = W2.shape[1]
    t_arr = jnp.asarray(temperature, jnp.float32).reshape(1, 1)

    grid = (n_tokens // TILE,)
    out = pl.pallas_call(
        partial(_router_tile, d_model=d_model),
        grid=grid,
        in_specs=[
            pl.BlockSpec(memory_space=pltpu.SMEM),
            pl.BlockSpec((TILE, d_model), lambda i: (i, 0)),
            pl.BlockSpec((TILE, n_feat), lambda i: (i, 0)),
            pl.BlockSpec((n_feat, d_femb), lambda i: (0, 0)),
            pl.BlockSpec((1, d_femb), lambda i: (0, 0)),
            pl.BlockSpec((d_model + d_femb, d_hid), lambda i: (0, 0)),
            pl.BlockSpec((1, d_hid), lambda i: (0, 0)),
            pl.BlockSpec((d_hid, n_stages), lambda i: (0, 0)),
            pl.BlockSpec((1, n_stages), lambda i: (0, 0)),
        ],
        out_specs=[
            pl.BlockSpec((TILE, n_stages), lambda i: (i, 0)),
            pl.BlockSpec((TILE, n_stages), lambda i: (i, 0)),
        ],
        out_shape=[
            jax.ShapeDtypeStruct((n_tokens, n_stages), jnp.float32),
            jax.ShapeDtypeStruct((n_tokens, n_stages), jnp.float32),
        ],
        compiler_params=pltpu.CompilerParams(
            dimension_semantics=("parallel",)),
    )(t_arr, hidden, feat, W_feat, b_feat.reshape(1, -1), W1,
      b1.reshape(1, -1), W2, b2.reshape(1, -1))
    return out[0], out[1]


# speedup vs baseline: 1.0596x; 1.0134x over previous
import jax
import jax.numpy as jnp
from jax.experimental import pallas as pl
from jax.experimental.pallas import tpu as pltpu
from functools import partial

TILE = 2048


def _router_tile(t_ref, hid_ref, feat_ref, wf_ref, bf_ref, w1_ref, b1_ref,
                 w2_ref, b2_ref, w_out_ref, l_out_ref, *, d_model):
    femb = jnp.dot(feat_ref[...], wf_ref[...],
                   preferred_element_type=jnp.float32) + bf_ref[...]
    acc = jnp.dot(femb, w1_ref[d_model:, :],
                  preferred_element_type=jnp.float32)
    acc = acc + jnp.dot(hid_ref[...], w1_ref[0:d_model, :],
                        preferred_element_type=jnp.float32)
    h = jnp.maximum(acc + b1_ref[...], 0.0)
    logits = jnp.dot(h, w2_ref[...],
                     preferred_element_type=jnp.float32) + b2_ref[...]
    l_out_ref[...] = logits

    scaled = logits.T / t_ref[0, 0]
    n_stages = scaled.shape[0]
    idx = jax.lax.broadcasted_iota(jnp.int32, scaled.shape, 0)
    m1 = jnp.max(scaled, axis=0, keepdims=True)
    first = jnp.min(jnp.where(scaled == m1, idx, n_stages), axis=0,
                    keepdims=True)
    m2 = jnp.max(jnp.where(idx == first, -jnp.inf, scaled), axis=0,
                 keepdims=True)
    keep = scaled >= m2
    e = jnp.where(keep, jnp.exp(scaled - m1), 0.0)
    w_out_ref[...] = (e / jnp.sum(e, axis=0, keepdims=True)).T


def kernel(hidden, feat, W_feat, b_feat, W1, b1, W2, b2, temperature):
    n_tokens, d_model = hidden.shape
    n_feat, d_femb = W_feat.shape
    d_hid = W1.shape[1]
    n_stages = W2.shape[1]
    t_arr = jnp.asarray(temperature, jnp.float32).reshape(1, 1)

    grid = (n_tokens // TILE,)
    out = pl.pallas_call(
        partial(_router_tile, d_model=d_model),
        grid=grid,
        in_specs=[
            pl.BlockSpec(memory_space=pltpu.SMEM),
            pl.BlockSpec((TILE, d_model), lambda i: (i, 0)),
            pl.BlockSpec((TILE, n_feat), lambda i: (i, 0)),
            pl.BlockSpec((n_feat, d_femb), lambda i: (0, 0)),
            pl.BlockSpec((1, d_femb), lambda i: (0, 0)),
            pl.BlockSpec((d_model + d_femb, d_hid), lambda i: (0, 0)),
            pl.BlockSpec((1, d_hid), lambda i: (0, 0)),
            pl.BlockSpec((d_hid, n_stages), lambda i: (0, 0)),
            pl.BlockSpec((1, n_stages), lambda i: (0, 0)),
        ],
        out_specs=[
            pl.BlockSpec((TILE, n_stages), lambda i: (i, 0)),
            pl.BlockSpec((TILE, n_stages), lambda i: (i, 0)),
        ],
        out_shape=[
            jax.ShapeDtypeStruct((n_tokens, n_stages), jnp.float32),
            jax.ShapeDtypeStruct((n_tokens, n_stages), jnp.float32),
        ],
        compiler_params=pltpu.CompilerParams(
            dimension_semantics=("parallel",)),
    )(t_arr, hidden, feat, W_feat, b_feat.reshape(1, -1), W1,
      b1.reshape(1, -1), W2, b2.reshape(1, -1))
    return out[0], out[1]
